# baseline (speedup 1.0000x reference)
import dataclasses
import functools

import jax
import jax.numpy as jnp
from jax import lax
from jax.experimental import pallas as pl
from jax.experimental.pallas import tpu as pltpu
from jax.experimental.pallas import tpu_sc as plsc

N_VOCAB = 128000
K = 2048
GROUP_SIZE = 128
G = K // GROUP_SIZE

NC = 2
NS = 16
NW = NC * NS

CH = 16


def _sc_gather(idx, qweights, szc):
    n_tok = idx.shape[0]
    assert n_tok % (NW * CH) == 0
    b_per_w = n_tok // NW
    n_ch = b_per_w // CH
    assert n_ch >= 2

    mesh = plsc.VectorSubcoreMesh(core_axis_name="c", subcore_axis_name="s")
    cp = pltpu.CompilerParams()
    if "needs_layout_passes" in pltpu.CompilerParams.__dataclass_fields__:
        cp = dataclasses.replace(cp, needs_layout_passes=False)

    @functools.partial(
        pl.kernel,
        mesh=mesh,
        compiler_params=cp,
        out_type=(
            jax.ShapeDtypeStruct((n_tok, K), jnp.int32),
            jax.ShapeDtypeStruct((n_tok, G), jnp.int32),
            jax.ShapeDtypeStruct((n_tok, G), jnp.int32),
        ),
        scratch_types=[
            pltpu.VMEM((CH,), jnp.int32),
            pltpu.VMEM((CH,), jnp.int32),
            pltpu.VMEM((CH,), jnp.int32),
            pltpu.VMEM((CH,), jnp.int32),
            pltpu.VMEM((CH, K), jnp.int32),
            pltpu.VMEM((CH, K), jnp.int32),
            pltpu.VMEM((CH, 128), jnp.int32),
            pltpu.VMEM((CH, 128), jnp.int32),
            pltpu.VMEM((CH, G), jnp.int32),
            pltpu.VMEM((CH, G), jnp.int32),
            pltpu.VMEM((CH, G), jnp.int32),
            pltpu.VMEM((CH, G), jnp.int32),
            pltpu.SemaphoreType.DMA,
            pltpu.SemaphoreType.DMA,
            pltpu.SemaphoreType.DMA,
            pltpu.SemaphoreType.DMA,
        ],
    )
    def k(idx_hbm, qw_hbm, szc_hbm, qg_hbm, sg_hbm, zg_hbm,
          ib0, ib1, gb0, gb1, qb0, qb1, szb0, szb1, sb0, sb1, zb0, zb1,
          gs0, gs1, ws0, ws1):
        wid = lax.axis_index("s") * NC + lax.axis_index("c")
        base = wid * b_per_w
        qw32 = qw_hbm.bitcast(jnp.int32)
        ibufs, gbufs = (ib0, ib1), (gb0, gb1)
        qbufs, szbufs = (qb0, qb1), (szb0, szb1)
        sbufs, zbufs = (sb0, sb1), (zb0, zb1)
        gsems, wsems = (gs0, gs1), (ws0, ws1)
        iota16 = lax.iota(jnp.int32, 16)

        def start_gather(c):
            b = c % 2
            off = base + c * CH
            pltpu.sync_copy(idx_hbm.at[pl.ds(off, CH)], ibufs[b])
            for i in range(CH // 16):
                sl = pl.ds(i * 16, 16)
                gbufs[b][sl] = ibufs[b][sl] >> 2
            return (
                pltpu.async_copy(qw32.at[gbufs[b]], qbufs[b], gsems[b]),
                pltpu.async_copy(szc_hbm.at[gbufs[b]], szbufs[b], gsems[b]),
            )

        def select_sz(c):
            b = c % 2

            @pl.loop(0, CH)
            def _(t):
                tvec = jnp.full((16,), t, jnp.int32)
                rvec = plsc.load_gather(ibufs[b], [tvec])
                pvec = ((rvec & 3) << 4) + iota16
                sbufs[b][t] = plsc.load_gather(szbufs[b], [tvec, pvec])
                zbufs[b][t] = plsc.load_gather(szbufs[b], [tvec, pvec + 64])

        def start_write(c):
            b = c % 2
            off = base + c * CH
            return (
                pltpu.async_copy(qbufs[b], qg_hbm.at[pl.ds(off, CH)], wsems[b]),
                pltpu.async_copy(sbufs[b], sg_hbm.at[pl.ds(off, CH)], wsems[b]),
                pltpu.async_copy(zbufs[b], zg_hbm.at[pl.ds(off, CH)], wsems[b]),
            )

        gh, wh = {}, {}
        gh[0] = start_gather(0)
        for c in range(n_ch):
            if c + 1 < n_ch:
                if c >= 1:
                    for h in wh[c - 1]:
                        h.wait()
                gh[c + 1] = start_gather(c + 1)
            for h in gh[c]:
                h.wait()
            select_sz(c)
            wh[c] = start_write(c)
        for c in (n_ch - 2, n_ch - 1):
            for h in wh[c]:
                h.wait()

    return k(idx, qweights, szc)


def _tc_dequant(flat, qg, sg, zg):
    n_tok = qg.shape[0]
    TB = 256
    assert n_tok % TB == 0

    q3 = qg.reshape(n_tok, G, GROUP_SIZE)
    s3 = sg.reshape(n_tok, G, 1)
    z3 = zg.reshape(n_tok, G, 1)
    x3 = flat.reshape(n_tok, 1, 1)

    def body(x_ref, q_ref, s_ref, z_ref, o_ref):
        w = q_ref[...]
        sh = 24 - ((x_ref[...] & 3) << 3)
        q = (w << sh) >> 24
        z = z_ref[...].astype(jnp.float32)
        s = s_ref[...]
        o_ref[...] = (q.astype(jnp.float32) - z) * s

    out = pl.pallas_call(
        body,
        grid=(n_tok // TB,),
        in_specs=[
            pl.BlockSpec((TB, 1, 1), lambda i: (i, 0, 0)),
            pl.BlockSpec((TB, G, GROUP_SIZE), lambda i: (i, 0, 0)),
            pl.BlockSpec((TB, G, 1), lambda i: (i, 0, 0)),
            pl.BlockSpec((TB, G, 1), lambda i: (i, 0, 0)),
        ],
        out_specs=pl.BlockSpec((TB, G, GROUP_SIZE), lambda i: (i, 0, 0)),
        out_shape=jax.ShapeDtypeStruct((n_tok, G, GROUP_SIZE), jnp.float32),
    )(x3, q3, s3, z3)
    return out


def kernel(x, qweights, scales, zeros):
    shape = x.shape
    flat = x.reshape(-1)
    s32 = lax.bitcast_convert_type(scales, jnp.int32)
    szc = jnp.concatenate(
        [s32.reshape(N_VOCAB // 4, 64), zeros.reshape(N_VOCAB // 4, 64)],
        axis=1,
    )
    qg, sg, zg = _sc_gather(flat, qweights, szc)
    out = _tc_dequant(flat, qg, lax.bitcast_convert_type(sg, jnp.float32), zg)
    return out.reshape(*shape, K)


# speedup vs baseline: 1.9087x; 1.9087x over previous
import dataclasses
import functools

import jax
import jax.numpy as jnp
from jax import lax
from jax.experimental import pallas as pl
from jax.experimental.pallas import tpu as pltpu
from jax.experimental.pallas import tpu_sc as plsc

N_VOCAB = 128000
K = 2048
GROUP_SIZE = 128
G = K // GROUP_SIZE

NC = 2
NS = 16
NW = NC * NS

CH = 16


def _sc_gather(idx, qweights, szc):
    n_tok = idx.shape[0]
    assert n_tok % (NW * CH) == 0
    b_per_w = n_tok // NW
    n_ch = b_per_w // CH
    assert n_ch >= 2

    mesh = plsc.VectorSubcoreMesh(core_axis_name="c", subcore_axis_name="s")
    cp = pltpu.CompilerParams()
    if "needs_layout_passes" in pltpu.CompilerParams.__dataclass_fields__:
        cp = dataclasses.replace(cp, needs_layout_passes=False)

    @functools.partial(
        pl.kernel,
        mesh=mesh,
        compiler_params=cp,
        out_type=(
            jax.ShapeDtypeStruct((n_tok, K), jnp.int32),
            jax.ShapeDtypeStruct((n_tok, 128), jnp.int32),
        ),
        scratch_types=[
            pltpu.VMEM((CH,), jnp.int32),
            pltpu.VMEM((CH,), jnp.int32),
            pltpu.VMEM((CH,), jnp.int32),
            pltpu.VMEM((CH,), jnp.int32),
            pltpu.VMEM((CH, K), jnp.int32),
            pltpu.VMEM((CH, K), jnp.int32),
            pltpu.VMEM((CH, 128), jnp.int32),
            pltpu.VMEM((CH, 128), jnp.int32),
            pltpu.SemaphoreType.DMA,
            pltpu.SemaphoreType.DMA,
            pltpu.SemaphoreType.DMA,
            pltpu.SemaphoreType.DMA,
        ],
    )
    def k(idx_hbm, qw_hbm, szc_hbm, qg_hbm, szp_hbm,
          ib0, ib1, gb0, gb1, qb0, qb1, szb0, szb1, gs0, gs1, ws0, ws1):
        wid = lax.axis_index("s") * NC + lax.axis_index("c")
        base = wid * b_per_w
        qw32 = qw_hbm.bitcast(jnp.int32)
        ibufs, gbufs = (ib0, ib1), (gb0, gb1)
        qbufs, szbufs = (qb0, qb1), (szb0, szb1)
        gsems, wsems = (gs0, gs1), (ws0, ws1)
        iota16 = lax.iota(jnp.int32, 16)

        def start_gather(c):
            b = c % 2
            off = base + c * CH
            pltpu.sync_copy(idx_hbm.at[pl.ds(off, CH)], ibufs[b])
            for i in range(CH // 16):
                sl = pl.ds(i * 16, 16)
                gbufs[b][sl] = ibufs[b][sl] >> 2
            return (
                pltpu.async_copy(qw32.at[gbufs[b]], qbufs[b], gsems[b]),
                pltpu.async_copy(szc_hbm.at[gbufs[b]], szbufs[b], gsems[b]),
            )

        def select_sz(c):
            b = c % 2

            @pl.loop(0, CH)
            def _(t):
                tvec = jnp.full((16,), t, jnp.int32)
                rvec = plsc.load_gather(ibufs[b], [tvec])
                pvec = ((rvec & 3) << 4) + iota16
                svals = plsc.load_gather(szbufs[b], [tvec, pvec])
                zvals = plsc.load_gather(szbufs[b], [tvec, pvec + 64])
                szbufs[b][t, pl.ds(0, 16)] = svals
                szbufs[b][t, pl.ds(16, 16)] = zvals
                szbufs[b][t, pl.ds(32, 16)] = rvec & 3

        def start_write(c):
            b = c % 2
            off = base + c * CH
            return (
                pltpu.async_copy(qbufs[b], qg_hbm.at[pl.ds(off, CH)], wsems[b]),
                pltpu.async_copy(szbufs[b], szp_hbm.at[pl.ds(off, CH)], wsems[b]),
            )

        gh, wh = {}, {}
        gh[0] = start_gather(0)
        for c in range(n_ch):
            if c + 1 < n_ch:
                if c >= 1:
                    for h in wh[c - 1]:
                        h.wait()
                gh[c + 1] = start_gather(c + 1)
            for h in gh[c]:
                h.wait()
            select_sz(c)
            wh[c] = start_write(c)
        for c in (n_ch - 2, n_ch - 1):
            for h in wh[c]:
                h.wait()

    return k(idx, qweights, szc)


def _tc_dequant(qg, szp):
    n_tok = qg.shape[0]
    TB = 256
    assert n_tok % TB == 0

    def body(q_ref, m_ref, o_ref):
        w = q_ref[...]
        m = m_ref[...]
        s16 = lax.bitcast_convert_type(m[:, 0:G], jnp.float32)
        z16 = m[:, G:2 * G].astype(jnp.float32)
        p = m[:, 2 * G:2 * G + 1]
        sh = 24 - (p << 3)
        q = ((w << sh) >> 24).astype(jnp.float32)
        lane_g = lax.broadcasted_iota(jnp.int32, (G, K), 1) >> 7
        row_g = lax.broadcasted_iota(jnp.int32, (G, K), 0)
        e = (lane_g == row_g).astype(jnp.float32)
        s_full = jnp.dot(s16, e, preferred_element_type=jnp.float32)
        zs_full = jnp.dot(z16 * s16, e, preferred_element_type=jnp.float32)
        o_ref[...] = q * s_full - zs_full

    out = pl.pallas_call(
        body,
        grid=(n_tok // TB,),
        in_specs=[
            pl.BlockSpec((TB, K), lambda i: (i, 0)),
            pl.BlockSpec((TB, 128), lambda i: (i, 0)),
        ],
        out_specs=pl.BlockSpec((TB, K), lambda i: (i, 0)),
        out_shape=jax.ShapeDtypeStruct((n_tok, K), jnp.float32),
    )(qg, szp)
    return out


def kernel(x, qweights, scales, zeros):
    shape = x.shape
    flat = x.reshape(-1)
    s32 = lax.bitcast_convert_type(scales, jnp.int32)
    szc = jnp.concatenate(
        [s32.reshape(N_VOCAB // 4, 64), zeros.reshape(N_VOCAB // 4, 64)],
        axis=1,
    )
    qg, szp = _sc_gather(flat, qweights, szc)
    out = _tc_dequant(qg, szp)
    return out.reshape(*shape, K)


# speedup vs baseline: 1.9122x; 1.0018x over previous
import dataclasses
import functools

import jax
import jax.numpy as jnp
from jax import lax
from jax.experimental import pallas as pl
from jax.experimental.pallas import tpu as pltpu
from jax.experimental.pallas import tpu_sc as plsc

N_VOCAB = 128000
K = 2048
GROUP_SIZE = 128
G = K // GROUP_SIZE

NC = 2
NS = 16
NW = NC * NS

CH = 16


def _sc_gather(idx, qweights, szc):
    n_tok = idx.shape[0]
    assert n_tok % (NW * CH) == 0
    b_per_w = n_tok // NW
    n_ch = b_per_w // CH
    assert n_ch >= 2

    mesh = plsc.VectorSubcoreMesh(core_axis_name="c", subcore_axis_name="s")
    cp = pltpu.CompilerParams()
    if "needs_layout_passes" in pltpu.CompilerParams.__dataclass_fields__:
        cp = dataclasses.replace(cp, needs_layout_passes=False)
    cp = dataclasses.replace(cp, use_tc_tiling_on_sc=True)

    @functools.partial(
        pl.kernel,
        mesh=mesh,
        compiler_params=cp,
        out_type=(
            jax.ShapeDtypeStruct((n_tok, K), jnp.int32),
            jax.ShapeDtypeStruct((n_tok, 128), jnp.int32),
        ),
        scratch_types=[
            pltpu.VMEM((CH,), jnp.int32),
            pltpu.VMEM((CH,), jnp.int32),
            pltpu.VMEM((CH,), jnp.int32),
            pltpu.VMEM((CH,), jnp.int32),
            pltpu.VMEM((CH, K), jnp.int32),
            pltpu.VMEM((CH, K), jnp.int32),
            pltpu.VMEM((CH, 128), jnp.int32),
            pltpu.VMEM((CH, 128), jnp.int32),
            pltpu.SemaphoreType.DMA,
            pltpu.SemaphoreType.DMA,
            pltpu.SemaphoreType.DMA,
            pltpu.SemaphoreType.DMA,
        ],
    )
    def k(idx_hbm, qw_hbm, szc_hbm, qg_hbm, szp_hbm,
          ib0, ib1, gb0, gb1, qb0, qb1, szb0, szb1, gs0, gs1, ws0, ws1):
        wid = lax.axis_index("s") * NC + lax.axis_index("c")
        base = wid * b_per_w
        qw32 = qw_hbm.bitcast(jnp.int32)
        ibufs, gbufs = (ib0, ib1), (gb0, gb1)
        qbufs, szbufs = (qb0, qb1), (szb0, szb1)
        gsems, wsems = (gs0, gs1), (ws0, ws1)
        iota16 = lax.iota(jnp.int32, 16)

        def start_gather(c):
            b = c % 2
            off = base + c * CH
            pltpu.sync_copy(idx_hbm.at[pl.ds(off, CH)], ibufs[b])
            for i in range(CH // 16):
                sl = pl.ds(i * 16, 16)
                gbufs[b][sl] = ibufs[b][sl] >> 2
            return (
                pltpu.async_copy(qw32.at[gbufs[b]], qbufs[b], gsems[b]),
                pltpu.async_copy(szc_hbm.at[gbufs[b]], szbufs[b], gsems[b]),
            )

        def select_sz(c):
            b = c % 2

            @pl.loop(0, CH)
            def _(t):
                tvec = jnp.full((16,), t, jnp.int32)
                rvec = plsc.load_gather(ibufs[b], [tvec])
                pvec = ((rvec & 3) << 4) + iota16
                svals = plsc.load_gather(szbufs[b], [tvec, pvec])
                zvals = plsc.load_gather(szbufs[b], [tvec, pvec + 64])
                szbufs[b][t, pl.ds(0, 16)] = svals
                szbufs[b][t, pl.ds(16, 16)] = zvals
                szbufs[b][t, pl.ds(32, 16)] = rvec & 3

        def start_write(c):
            b = c % 2
            off = base + c * CH
            return (
                pltpu.async_copy(qbufs[b], qg_hbm.at[pl.ds(off, CH)], wsems[b]),
                pltpu.async_copy(szbufs[b], szp_hbm.at[pl.ds(off, CH)], wsems[b]),
            )

        gh, wh = {}, {}
        gh[0] = start_gather(0)
        for c in range(n_ch):
            if c + 1 < n_ch:
                if c >= 1:
                    for h in wh[c - 1]:
                        h.wait()
                gh[c + 1] = start_gather(c + 1)
            for h in gh[c]:
                h.wait()
            select_sz(c)
            wh[c] = start_write(c)
        for c in (n_ch - 2, n_ch - 1):
            for h in wh[c]:
                h.wait()

    return k(idx, qweights, szc)


def _tc_dequant(qg, szp):
    n_tok = qg.shape[0]
    TB = 256
    assert n_tok % TB == 0

    def body(q_ref, m_ref, o_ref):
        w = q_ref[...]
        m = m_ref[...]
        s16 = lax.bitcast_convert_type(m[:, 0:G], jnp.float32)
        z16 = m[:, G:2 * G].astype(jnp.float32)
        p = m[:, 2 * G:2 * G + 1]
        sh = 24 - (p << 3)
        q = ((w << sh) >> 24).astype(jnp.float32)
        lane_g = lax.broadcasted_iota(jnp.int32, (G, K), 1) >> 7
        row_g = lax.broadcasted_iota(jnp.int32, (G, K), 0)
        e = (lane_g == row_g).astype(jnp.float32)
        s_full = jnp.dot(s16, e, preferred_element_type=jnp.float32)
        zs_full = jnp.dot(z16 * s16, e, preferred_element_type=jnp.float32)
        o_ref[...] = q * s_full - zs_full

    out = pl.pallas_call(
        body,
        grid=(n_tok // TB,),
        in_specs=[
            pl.BlockSpec((TB, K), lambda i: (i, 0)),
            pl.BlockSpec((TB, 128), lambda i: (i, 0)),
        ],
        out_specs=pl.BlockSpec((TB, K), lambda i: (i, 0)),
        out_shape=jax.ShapeDtypeStruct((n_tok, K), jnp.float32),
    )(qg, szp)
    return out


def kernel(x, qweights, scales, zeros):
    shape = x.shape
    flat = x.reshape(-1)
    s32 = lax.bitcast_convert_type(scales, jnp.int32)
    szc = jnp.concatenate(
        [s32.reshape(N_VOCAB // 4, 64), zeros.reshape(N_VOCAB // 4, 64)],
        axis=1,
    )
    qg, szp = _sc_gather(flat, qweights, szc)
    out = _tc_dequant(qg, szp)
    return out.reshape(*shape, K)
